# Initial kernel scaffold; baseline (speedup 1.0000x reference)
import jax
import jax.numpy as jnp
from jax.experimental import pallas as pl


def kernel(x, w_conv, bn_scale, bn_shift, w_reduce, w_pred1, w_pred2, t_scale, fuse_w):
    raise NotImplementedError("write your pallas kernel here")


# speedup vs baseline: 1.3369x; 1.3369x over previous
import numpy as np
import jax
import jax.numpy as jnp
from jax.experimental import pallas as pl
from jax.experimental.pallas import tpu as pltpu


def _shift(v, s):
    if s > 0:
        return jnp.concatenate([jnp.zeros((1, s), v.dtype), v[:, :-s]], axis=1)
    return jnp.concatenate([v[:, -s:], jnp.zeros((1, -s), v.dtype)], axis=1)


def _fused_kernel(H, W):
    HW = H * W

    def body(x_ref, w1s_ref, shift_ref, wbar_ref, wrt_ref, wp1t_ref,
             wp2t_ref, dtc_ref, dt2c_ref, fuse_ref, ml_ref, mr_ref, out_ref):
        x = x_ref[0]
        z = jnp.dot(w1s_ref[...], x, preferred_element_type=jnp.float32)
        z = z + shift_ref[...]
        z = z * jax.nn.sigmoid(z)

        zbar = jnp.dot(wbar_ref[...], z, preferred_element_type=jnp.float32)
        zsum = jnp.sum(z, axis=1, keepdims=True)

        y = jnp.dot(wrt_ref[...], zsum,
                    preferred_element_type=jnp.float32) * (1.0 / HW)
        h1 = jax.nn.relu(jnp.dot(wp1t_ref[...], y,
                                 preferred_element_type=jnp.float32))
        logits = jnp.dot(wp2t_ref[...], h1,
                         preferred_element_type=jnp.float32)
        m = jnp.max(logits, axis=0, keepdims=True)
        e = jnp.exp(logits - m)
        den = jnp.sum(e, axis=0, keepdims=True)
        ca = -jnp.sum(e * dtc_ref[...], axis=0, keepdims=True) / den
        cb = 0.5 * jnp.sum(e * dt2c_ref[...], axis=0, keepdims=True) / den

        ml = ml_ref[...]
        mr = mr_ref[...]

        def lap(v):
            return (_shift(v, W) + _shift(v, -W)
                    + ml * _shift(v, 1) + mr * _shift(v, -1) - 4.0 * v)

        l1 = lap(zbar)
        l2 = lap(l1)
        acc = ca * l1 + cb * l2
        amap = jax.nn.sigmoid(fuse_ref[...] * acc)

        out_ref[0] = z * (1.0 + amap)

    return body


@jax.jit
def kernel(x, w_conv, bn_scale, bn_shift, w_reduce, w_pred1, w_pred2,
           t_scale, fuse_w):
    B, C1, H, W = x.shape
    HW = H * W
    C2 = w_conv.shape[1]
    CR = w_reduce.shape[1]
    HID = w_pred1.shape[1]
    T = w_pred2.shape[1]

    x3 = x.reshape(B, C1, HW)

    w1s = w_conv.T * bn_scale[:, None]
    shift = bn_shift.reshape(C2, 1)
    wbar = w_reduce.mean(axis=1).reshape(1, C2)
    wrt = w_reduce.T
    wp1t = w_pred1.T
    wp2t = w_pred2.T
    dt = t_scale[1:] - t_scale[:-1]
    dt2 = t_scale[1:] * t_scale[1:] - t_scale[:-1] * t_scale[:-1]
    dtc = jnp.concatenate([dt, jnp.zeros((1,), jnp.float32)]).reshape(T, 1)
    dt2c = jnp.concatenate([dt2, jnp.zeros((1,), jnp.float32)]).reshape(T, 1)
    fuse = fuse_w.reshape(1, 1)

    widx = np.arange(HW, dtype=np.int64) % W
    ml = jnp.asarray((widx != 0).astype(np.float32).reshape(1, HW))
    mr = jnp.asarray((widx != W - 1).astype(np.float32).reshape(1, HW))

    const = lambda *_: (0, 0)
    out3 = pl.pallas_call(
        _fused_kernel(H, W),
        out_shape=jax.ShapeDtypeStruct((B, C2, HW), jnp.float32),
        grid=(B,),
        in_specs=[
            pl.BlockSpec((1, C1, HW), lambda b: (b, 0, 0)),
            pl.BlockSpec((C2, C1), const),
            pl.BlockSpec((C2, 1), const),
            pl.BlockSpec((1, C2), const),
            pl.BlockSpec((CR, C2), const),
            pl.BlockSpec((HID, CR), const),
            pl.BlockSpec((T, HID), const),
            pl.BlockSpec((T, 1), const),
            pl.BlockSpec((T, 1), const),
            pl.BlockSpec((1, 1), const),
            pl.BlockSpec((1, HW), const),
            pl.BlockSpec((1, HW), const),
        ],
        out_specs=pl.BlockSpec((1, C2, HW), lambda b: (b, 0, 0)),
        compiler_params=pltpu.CompilerParams(
            dimension_semantics=("parallel",),
            vmem_limit_bytes=48 << 20),
    )(x3, w1s, shift, wbar, wrt, wp1t, wp2t, dtc, dt2c, fuse, ml, mr)

    return out3.reshape(B, C2, H, W)
